# baseline (speedup 1.0000x reference)
import functools

import jax
import jax.numpy as jnp
from jax import lax
from jax.experimental import pallas as pl
from jax.experimental.pallas import tpu as pltpu
from jax.experimental.pallas import tpu_sc as plsc

N = 10000
E = 160000
D = 256
HD = 128
B = 128
NS = 16
BPT = 80
EP = NS * BPT * B
NACC = N + 8
CH = 80
NCH = N // CH
CH_PER_TILE = (NCH + NS - 1) // NS
CW = 128


def _zero_fill(zbuf, width):
    z16 = jnp.zeros((16,), jnp.float32)

    def zrow(i, carry):
        for j in range(width // 16):
            zbuf[i, pl.ds(j * 16, 16)] = z16
        return carry

    lax.fori_loop(0, CH, zrow, 0)


def _for_row_chunks(s, body_fn):
    def f(j, carry):
        k = j * NS + s

        @pl.when(k < NCH)
        def _():
            body_fn(pl.multiple_of(k * CH, CH))

        return carry

    lax.fori_loop(0, CH_PER_TILE, f, 0)


_HB = BPT // 2


def _segsum_body(y_h, row_h, col_h, out, acc, rowi, coli, buf0,
                 sem0, sem1):
    c = lax.axis_index("c")
    s = lax.axis_index("s")
    yoff = c * N

    _zero_fill(buf0, HD)
    _for_row_chunks(
        s, lambda r0: pltpu.sync_copy(buf0.at[pl.ds(0, CH)],
                                      acc.at[pl.ds(r0, CH)]))
    plsc.subcore_barrier()

    def blk(j, carry):
        e0 = pl.multiple_of((s * BPT + j) * B, B)
        pltpu.sync_copy(row_h.at[pl.ds(e0, B)], rowi)
        pltpu.sync_copy(col_h.at[pl.ds(e0, B)], coli)
        for k in range(B // 16):
            rowi[pl.ds(k * 16, 16)] = rowi[pl.ds(k * 16, 16)] + yoff
        pltpu.async_copy(y_h.at[rowi], buf0, sem0).wait()
        pltpu.sync_copy(buf0, acc.at[coli], add=True)
        return carry

    lax.fori_loop(0, BPT, blk, 0)
    plsc.subcore_barrier()

    def wb(r0):
        pltpu.sync_copy(acc.at[pl.ds(r0, CH)], buf0.at[pl.ds(0, CH)])
        pltpu.sync_copy(buf0.at[pl.ds(0, CH)], out.at[pl.ds(yoff + r0, CH)])

    _for_row_chunks(s, wb)


@functools.cache
def _get_segsum():
    return pl.kernel(
        _segsum_body,
        out_type=jax.ShapeDtypeStruct((2 * N, HD), jnp.float32),
        mesh=plsc.VectorSubcoreMesh(core_axis_name="c", subcore_axis_name="s"),
        scratch_types=[
            pltpu.VMEM_SHARED((NACC, HD), jnp.float32),
            pltpu.VMEM((B,), jnp.int32),
            pltpu.VMEM((B,), jnp.int32),
            pltpu.VMEM((B, HD), jnp.float32),
            pltpu.SemaphoreType.DMA,
            pltpu.SemaphoreType.DMA,
        ],
    )


def _counts_body(cols_h, out, acc, colb, ones, zbuf, sem0, sem1):
    c = lax.axis_index("c")
    s = lax.axis_index("s")
    yoff = c * N

    t0 = pl.multiple_of(c * (EP // B) + s * BPT, BPT)
    pltpu.sync_copy(cols_h.at[pl.ds(t0, BPT)], colb)

    one16 = jnp.ones((16,), jnp.float32)

    def orow(i, carry):
        for j in range(CW // 16):
            ones[i, pl.ds(j * 16, 16)] = one16
        return carry

    lax.fori_loop(0, B, orow, 0)

    _zero_fill(zbuf, CW)
    _for_row_chunks(s, lambda r0: pltpu.sync_copy(zbuf, acc.at[pl.ds(r0, CH)]))
    plsc.subcore_barrier()

    def sstart(j, sem):
        pltpu.async_copy(ones, acc.at[colb.at[j]], sem, add=True)

    def sdrain(sem):
        pltpu.make_async_copy(out.at[pl.ds(0, B)], ones, sem).wait()

    sstart(0, sem0)

    def blk(g, carry):
        j0 = g * 2
        sstart(j0 + 1, sem1)
        sdrain(sem0)

        @pl.when(g < BPT // 2 - 1)
        def _():
            sstart(j0 + 2, sem0)

        sdrain(sem1)
        return carry

    lax.fori_loop(0, BPT // 2, blk, 0)
    plsc.subcore_barrier()

    def wb(r0):
        pltpu.sync_copy(acc.at[pl.ds(r0, CH)], zbuf)
        pltpu.sync_copy(zbuf, out.at[pl.ds(yoff + r0, CH)])

    _for_row_chunks(s, wb)


@functools.cache
def _get_counts():
    return pl.kernel(
        _counts_body,
        out_type=jax.ShapeDtypeStruct((2 * N, CW), jnp.float32),
        mesh=plsc.VectorSubcoreMesh(core_axis_name="c", subcore_axis_name="s"),
        scratch_types=[
            pltpu.VMEM_SHARED((NACC, CW), jnp.float32),
            pltpu.VMEM((BPT, B), jnp.int32),
            pltpu.VMEM((B, CW), jnp.float32),
            pltpu.VMEM((CH, CW), jnp.float32),
            pltpu.SemaphoreType.DMA,
            pltpu.SemaphoreType.DMA,
        ],
    )


_R = 1000
_CONTRACT = (((1,), (1,)), ((), ()))


def _tca_body(xa_ref, xp_ref, wl1_ref, wr1_ref, wr2_ref, bl1_ref, bl2_ref,
              y1_ref, z1_ref, z2_ref):
    xa = xa_ref[...]
    y1 = lax.dot_general(xa, wl1_ref[...], _CONTRACT,
                         preferred_element_type=jnp.float32)
    y1_ref[0] = y1[:, :HD]
    y1_ref[1] = y1[:, HD:]
    z1_ref[...] = lax.dot_general(xp_ref[...], wr1_ref[...], _CONTRACT,
                                  preferred_element_type=jnp.float32) + bl1_ref[...]
    z2_ref[...] = lax.dot_general(xa, wr2_ref[...], _CONTRACT,
                                  preferred_element_type=jnp.float32) + bl2_ref[...]


_tc_a = pl.pallas_call(
    _tca_body,
    grid=(N // _R,),
    in_specs=[
        pl.BlockSpec((_R, D), lambda i: (i, 0)),
        pl.BlockSpec((_R, D), lambda i: (i, 0)),
        pl.BlockSpec((D, D), lambda i: (0, 0)),
        pl.BlockSpec((D, D), lambda i: (0, 0)),
        pl.BlockSpec((D, D), lambda i: (0, 0)),
        pl.BlockSpec((1, D), lambda i: (0, 0)),
        pl.BlockSpec((1, D), lambda i: (0, 0)),
    ],
    out_specs=[
        pl.BlockSpec((2, _R, HD), lambda i: (0, i, 0)),
        pl.BlockSpec((_R, D), lambda i: (i, 0)),
        pl.BlockSpec((_R, D), lambda i: (i, 0)),
    ],
    out_shape=[
        jax.ShapeDtypeStruct((2, N, HD), jnp.float32),
        jax.ShapeDtypeStruct((N, D), jnp.float32),
        jax.ShapeDtypeStruct((N, D), jnp.float32),
    ],
)


def _tcb_body(s_ref, cnt_ref, z1_ref, wl2_ref, y2_ref):
    inv = 1.0 / jnp.maximum(cnt_ref[...], 1.0)
    h = jnp.concatenate([s_ref[0], s_ref[1]], axis=1) * inv + z1_ref[...]
    h = jnp.maximum(h, 0.0)
    y2 = lax.dot_general(h, wl2_ref[...], _CONTRACT,
                         preferred_element_type=jnp.float32)
    y2_ref[0] = y2[:, :HD]
    y2_ref[1] = y2[:, HD:]


_tc_b = pl.pallas_call(
    _tcb_body,
    grid=(N // _R,),
    in_specs=[
        pl.BlockSpec((2, _R, HD), lambda i: (0, i, 0)),
        pl.BlockSpec((_R, 1), lambda i: (i, 0)),
        pl.BlockSpec((_R, D), lambda i: (i, 0)),
        pl.BlockSpec((D, D), lambda i: (0, 0)),
    ],
    out_specs=pl.BlockSpec((2, _R, HD), lambda i: (0, i, 0)),
    out_shape=jax.ShapeDtypeStruct((2, N, HD), jnp.float32),
)


def _tcc_body(s_ref, cnt_ref, z2_ref, wo_ref, bo_ref, out_ref):
    inv = 1.0 / jnp.maximum(cnt_ref[...], 1.0)
    h = jnp.concatenate([s_ref[0], s_ref[1]], axis=1) * inv + z2_ref[...]
    h = jnp.maximum(h, 0.0)
    out_ref[...] = lax.dot_general(h, wo_ref[...], _CONTRACT,
                                   preferred_element_type=jnp.float32) + bo_ref[...]


_tc_c = pl.pallas_call(
    _tcc_body,
    grid=(N // _R,),
    in_specs=[
        pl.BlockSpec((2, _R, HD), lambda i: (0, i, 0)),
        pl.BlockSpec((_R, 1), lambda i: (i, 0)),
        pl.BlockSpec((_R, D), lambda i: (i, 0)),
        pl.BlockSpec((D, D), lambda i: (0, 0)),
        pl.BlockSpec((1, D), lambda i: (0, 0)),
    ],
    out_specs=pl.BlockSpec((_R, D), lambda i: (i, 0)),
    out_shape=jax.ShapeDtypeStruct((N, D), jnp.float32),
)


def _pad_idx(idx, fill):
    pad = jnp.full((EP - E,), fill, jnp.int32)
    return jnp.concatenate([idx, pad])


def kernel(x_author, x_paper, edge_index_writes, edge_index_written_by,
           Wl1, bl1, Wr1, Wl2, bl2, Wr2, Wo, bo):
    row1 = _pad_idx(edge_index_writes[0], 0)
    col1 = _pad_idx(edge_index_writes[1], N)
    row2 = _pad_idx(edge_index_written_by[0], 0)
    col2 = _pad_idx(edge_index_written_by[1], N)
    cols = jnp.concatenate([col1, col2]).reshape(2 * EP // B, B)

    cntw = _get_counts()(cols)
    cnt1, cnt2 = cntw[:N, :1], cntw[N:, :1]

    y1s, z1, z2 = _tc_a(x_author, x_paper, Wl1, Wr1, Wr2,
                        bl1.reshape(1, D), bl2.reshape(1, D))
    s1 = _get_segsum()(y1s.reshape(2 * N, HD), row1, col1)
    y2s = _tc_b(s1.reshape(2, N, HD), cnt1, z1, Wl2)
    s2 = _get_segsum()(y2s.reshape(2 * N, HD), row2, col2)
    return _tc_c(s2.reshape(2, N, HD), cnt2, z2, Wo, bo.reshape(1, D))


# speedup vs baseline: 1.6474x; 1.6474x over previous
import functools

import jax
import jax.numpy as jnp
from jax import lax
from jax.experimental import pallas as pl
from jax.experimental.pallas import tpu as pltpu
from jax.experimental.pallas import tpu_sc as plsc

N = 10000
E = 160000
D = 256
HD = 128
B = 128
NS = 16
BPT = 80
EP = NS * BPT * B
NACC = N + 8
CH = 80
NCH = N // CH
CH_PER_TILE = (NCH + NS - 1) // NS
CW = 128


def _zero_fill(zbuf, width):
    z16 = jnp.zeros((16,), jnp.float32)

    def zrow(i, carry):
        for j in range(width // 16):
            zbuf[i, pl.ds(j * 16, 16)] = z16
        return carry

    lax.fori_loop(0, CH, zrow, 0)


def _for_row_chunks(s, body_fn):
    def f(j, carry):
        k = j * NS + s

        @pl.when(k < NCH)
        def _():
            body_fn(pl.multiple_of(k * CH, CH))

        return carry

    lax.fori_loop(0, CH_PER_TILE, f, 0)


_HB = BPT // 2


NBLK = E // B
BLK_PER_TILE = (NBLK + NS - 1) // NS


def _segsum_body(y_h, row_h, col_h, out, acc, rowi, coli, buf, zbuf, sem):
    c = lax.axis_index("c")
    s = lax.axis_index("s")
    yoff = c * N

    _zero_fill(zbuf, HD)
    _for_row_chunks(s, lambda r0: pltpu.sync_copy(zbuf, acc.at[pl.ds(r0, CH)]))
    plsc.subcore_barrier()

    def blk(j, carry):
        b = j * NS + s

        @pl.when(b < NBLK)
        def _():
            e0 = pl.multiple_of(b * B, B)
            pltpu.sync_copy(row_h.at[pl.ds(e0, B)], rowi)
            pltpu.sync_copy(col_h.at[pl.ds(e0, B)], coli)
            for k in range(B // 16):
                rowi[pl.ds(k * 16, 16)] = rowi[pl.ds(k * 16, 16)] + yoff
            pltpu.async_copy(y_h.at[rowi], buf, sem).wait()
            pltpu.sync_copy(buf, acc.at[coli], add=True)

        return carry

    lax.fori_loop(0, BLK_PER_TILE, blk, 0)
    plsc.subcore_barrier()

    def wb(r0):
        pltpu.sync_copy(acc.at[pl.ds(r0, CH)], zbuf)
        pltpu.sync_copy(zbuf, out.at[pl.ds(yoff + r0, CH)])

    _for_row_chunks(s, wb)


@functools.cache
def _get_segsum():
    return pl.kernel(
        _segsum_body,
        out_type=jax.ShapeDtypeStruct((2 * N, HD), jnp.float32),
        mesh=plsc.VectorSubcoreMesh(core_axis_name="c", subcore_axis_name="s"),
        scratch_types=[
            pltpu.VMEM_SHARED((N, HD), jnp.float32),
            pltpu.VMEM((B,), jnp.int32),
            pltpu.VMEM((B,), jnp.int32),
            pltpu.VMEM((B, HD), jnp.float32),
            pltpu.VMEM((CH, HD), jnp.float32),
            pltpu.SemaphoreType.DMA,
        ],
    )


def _counts_body(cols_h, out, acc, colb, ones, zbuf, sem0, sem1):
    c = lax.axis_index("c")
    s = lax.axis_index("s")
    yoff = c * N

    t0 = pl.multiple_of(c * (EP // B) + s * BPT, BPT)
    pltpu.sync_copy(cols_h.at[pl.ds(t0, BPT)], colb)

    one16 = jnp.ones((16,), jnp.float32)

    def orow(i, carry):
        for j in range(CW // 16):
            ones[i, pl.ds(j * 16, 16)] = one16
        return carry

    lax.fori_loop(0, B, orow, 0)

    _zero_fill(zbuf, CW)
    _for_row_chunks(s, lambda r0: pltpu.sync_copy(zbuf, acc.at[pl.ds(r0, CH)]))
    plsc.subcore_barrier()

    def sstart(j, sem):
        pltpu.async_copy(ones, acc.at[colb.at[j]], sem, add=True)

    def sdrain(sem):
        pltpu.make_async_copy(out.at[pl.ds(0, B)], ones, sem).wait()

    sstart(0, sem0)

    def blk(g, carry):
        j0 = g * 2
        sstart(j0 + 1, sem1)
        sdrain(sem0)

        @pl.when(g < BPT // 2 - 1)
        def _():
            sstart(j0 + 2, sem0)

        sdrain(sem1)
        return carry

    lax.fori_loop(0, BPT // 2, blk, 0)
    plsc.subcore_barrier()

    def wb(r0):
        pltpu.sync_copy(acc.at[pl.ds(r0, CH)], zbuf)
        pltpu.sync_copy(zbuf, out.at[pl.ds(yoff + r0, CH)])

    _for_row_chunks(s, wb)


@functools.cache
def _get_counts():
    return pl.kernel(
        _counts_body,
        out_type=jax.ShapeDtypeStruct((2 * N, CW), jnp.float32),
        mesh=plsc.VectorSubcoreMesh(core_axis_name="c", subcore_axis_name="s"),
        scratch_types=[
            pltpu.VMEM_SHARED((NACC, CW), jnp.float32),
            pltpu.VMEM((BPT, B), jnp.int32),
            pltpu.VMEM((B, CW), jnp.float32),
            pltpu.VMEM((CH, CW), jnp.float32),
            pltpu.SemaphoreType.DMA,
            pltpu.SemaphoreType.DMA,
        ],
    )


_R = 1000
_CONTRACT = (((1,), (1,)), ((), ()))


def _tca_body(xa_ref, xp_ref, wl1_ref, wr1_ref, wr2_ref, bl1_ref, bl2_ref,
              y1_ref, z1_ref, z2_ref):
    xa = xa_ref[...]
    y1 = lax.dot_general(xa, wl1_ref[...], _CONTRACT,
                         preferred_element_type=jnp.float32)
    y1_ref[0] = y1[:, :HD]
    y1_ref[1] = y1[:, HD:]
    z1_ref[...] = lax.dot_general(xp_ref[...], wr1_ref[...], _CONTRACT,
                                  preferred_element_type=jnp.float32) + bl1_ref[...]
    z2_ref[...] = lax.dot_general(xa, wr2_ref[...], _CONTRACT,
                                  preferred_element_type=jnp.float32) + bl2_ref[...]


_tc_a = pl.pallas_call(
    _tca_body,
    grid=(N // _R,),
    in_specs=[
        pl.BlockSpec((_R, D), lambda i: (i, 0)),
        pl.BlockSpec((_R, D), lambda i: (i, 0)),
        pl.BlockSpec((D, D), lambda i: (0, 0)),
        pl.BlockSpec((D, D), lambda i: (0, 0)),
        pl.BlockSpec((D, D), lambda i: (0, 0)),
        pl.BlockSpec((1, D), lambda i: (0, 0)),
        pl.BlockSpec((1, D), lambda i: (0, 0)),
    ],
    out_specs=[
        pl.BlockSpec((2, _R, HD), lambda i: (0, i, 0)),
        pl.BlockSpec((_R, D), lambda i: (i, 0)),
        pl.BlockSpec((_R, D), lambda i: (i, 0)),
    ],
    out_shape=[
        jax.ShapeDtypeStruct((2, N, HD), jnp.float32),
        jax.ShapeDtypeStruct((N, D), jnp.float32),
        jax.ShapeDtypeStruct((N, D), jnp.float32),
    ],
)


def _tcb_body(s_ref, cnt_ref, z1_ref, wl2_ref, y2_ref):
    inv = 1.0 / jnp.maximum(cnt_ref[...], 1.0)
    h = jnp.concatenate([s_ref[0], s_ref[1]], axis=1) * inv + z1_ref[...]
    h = jnp.maximum(h, 0.0)
    y2 = lax.dot_general(h, wl2_ref[...], _CONTRACT,
                         preferred_element_type=jnp.float32)
    y2_ref[0] = y2[:, :HD]
    y2_ref[1] = y2[:, HD:]


_tc_b = pl.pallas_call(
    _tcb_body,
    grid=(N // _R,),
    in_specs=[
        pl.BlockSpec((2, _R, HD), lambda i: (0, i, 0)),
        pl.BlockSpec((_R, 1), lambda i: (i, 0)),
        pl.BlockSpec((_R, D), lambda i: (i, 0)),
        pl.BlockSpec((D, D), lambda i: (0, 0)),
    ],
    out_specs=pl.BlockSpec((2, _R, HD), lambda i: (0, i, 0)),
    out_shape=jax.ShapeDtypeStruct((2, N, HD), jnp.float32),
)


def _tcc_body(s_ref, cnt_ref, z2_ref, wo_ref, bo_ref, out_ref):
    inv = 1.0 / jnp.maximum(cnt_ref[...], 1.0)
    h = jnp.concatenate([s_ref[0], s_ref[1]], axis=1) * inv + z2_ref[...]
    h = jnp.maximum(h, 0.0)
    out_ref[...] = lax.dot_general(h, wo_ref[...], _CONTRACT,
                                   preferred_element_type=jnp.float32) + bo_ref[...]


_tc_c = pl.pallas_call(
    _tcc_body,
    grid=(N // _R,),
    in_specs=[
        pl.BlockSpec((2, _R, HD), lambda i: (0, i, 0)),
        pl.BlockSpec((_R, 1), lambda i: (i, 0)),
        pl.BlockSpec((_R, D), lambda i: (i, 0)),
        pl.BlockSpec((D, D), lambda i: (0, 0)),
        pl.BlockSpec((1, D), lambda i: (0, 0)),
    ],
    out_specs=pl.BlockSpec((_R, D), lambda i: (i, 0)),
    out_shape=jax.ShapeDtypeStruct((N, D), jnp.float32),
)


def _pad_idx(idx, fill):
    pad = jnp.full((EP - E,), fill, jnp.int32)
    return jnp.concatenate([idx, pad])


def kernel(x_author, x_paper, edge_index_writes, edge_index_written_by,
           Wl1, bl1, Wr1, Wl2, bl2, Wr2, Wo, bo):
    row1, col1 = edge_index_writes[0], edge_index_writes[1]
    row2, col2 = edge_index_written_by[0], edge_index_written_by[1]
    cols = jnp.concatenate([_pad_idx(col1, N), _pad_idx(col2, N)]
                           ).reshape(2 * EP // B, B)

    cntw = _get_counts()(cols)
    cnt1, cnt2 = cntw[:N, :1], cntw[N:, :1]

    y1s, z1, z2 = _tc_a(x_author, x_paper, Wl1, Wr1, Wr2,
                        bl1.reshape(1, D), bl2.reshape(1, D))
    s1 = _get_segsum()(y1s.reshape(2 * N, HD), row1, col1)
    y2s = _tc_b(s1.reshape(2, N, HD), cnt1, z1, Wl2)
    s2 = _get_segsum()(y2s.reshape(2 * N, HD), row2, col2)
    return _tc_c(s2.reshape(2, N, HD), cnt2, z2, Wo, bo.reshape(1, D))


# speedup vs baseline: 2.1337x; 1.2952x over previous
import functools

import jax
import jax.numpy as jnp
from jax import lax
from jax.experimental import pallas as pl
from jax.experimental.pallas import tpu as pltpu
from jax.experimental.pallas import tpu_sc as plsc

N = 10000
E = 160000
D = 256
HD = 128
B = 128
NS = 16
BPT = 80
EP = NS * BPT * B
NACC = N + 8
CH = 80
NCH = N // CH
CH_PER_TILE = (NCH + NS - 1) // NS
CW = 128


def _zero_fill(zbuf, width):
    z16 = jnp.zeros((16,), jnp.float32)

    def zrow(i, carry):
        for j in range(width // 16):
            zbuf[i, pl.ds(j * 16, 16)] = z16
        return carry

    lax.fori_loop(0, CH, zrow, 0)


def _for_row_chunks(s, body_fn):
    def f(j, carry):
        k = j * NS + s

        @pl.when(k < NCH)
        def _():
            body_fn(pl.multiple_of(k * CH, CH))

        return carry

    lax.fori_loop(0, CH_PER_TILE, f, 0)


_HB = BPT // 2


NBLK = E // B
BLK_PER_TILE = (NBLK + NS - 1) // NS


def _segsum_body(y_h, row_h, col_h, out, acc, rowi, coli, buf,
                 rowi1, coli1, buf1, zbuf, sem, sem1):
    c = lax.axis_index("c")
    s = lax.axis_index("s")
    yoff = c * N

    _zero_fill(zbuf, HD)
    _for_row_chunks(s, lambda r0: pltpu.sync_copy(zbuf, acc.at[pl.ds(r0, CH)]))
    plsc.subcore_barrier()

    def fetch(j, rowi_n, coli_n, buf_n, sem_n):
        e0 = pl.multiple_of((j * NS + s) * B, B)
        pltpu.sync_copy(row_h.at[pl.ds(e0, B)], rowi_n)
        pltpu.sync_copy(col_h.at[pl.ds(e0, B)], coli_n)
        for k in range(B // 16):
            rowi_n[pl.ds(k * 16, 16)] = rowi_n[pl.ds(k * 16, 16)] + yoff
        return pltpu.async_copy(y_h.at[rowi_n], buf_n, sem_n)

    def blk2(g, carry):
        d0 = fetch(2 * g, rowi, coli, buf, sem)
        d1 = fetch(2 * g + 1, rowi1, coli1, buf1, sem1)
        d0.wait()
        pltpu.sync_copy(buf, acc.at[coli], add=True)
        d1.wait()
        pltpu.sync_copy(buf1, acc.at[coli1], add=True)
        return carry

    lax.fori_loop(0, (BLK_PER_TILE - 1) // 2, blk2, 0)

    @pl.when(78 * NS + s < NBLK)
    def _():
        fetch(78, rowi, coli, buf, sem).wait()
        pltpu.sync_copy(buf, acc.at[coli], add=True)

    plsc.subcore_barrier()

    def wb(r0):
        pltpu.sync_copy(acc.at[pl.ds(r0, CH)], zbuf)
        pltpu.sync_copy(zbuf, out.at[pl.ds(yoff + r0, CH)])

    _for_row_chunks(s, wb)


@functools.cache
def _get_segsum():
    return pl.kernel(
        _segsum_body,
        out_type=jax.ShapeDtypeStruct((2 * N, HD), jnp.float32),
        mesh=plsc.VectorSubcoreMesh(core_axis_name="c", subcore_axis_name="s"),
        scratch_types=[
            pltpu.VMEM_SHARED((N, HD), jnp.float32),
            pltpu.VMEM((B,), jnp.int32),
            pltpu.VMEM((B,), jnp.int32),
            pltpu.VMEM((B, HD), jnp.float32),
            pltpu.VMEM((B,), jnp.int32),
            pltpu.VMEM((B,), jnp.int32),
            pltpu.VMEM((B, HD), jnp.float32),
            pltpu.VMEM((CH, HD), jnp.float32),
            pltpu.SemaphoreType.DMA,
            pltpu.SemaphoreType.DMA,
        ],
    )


def _counts_body(cols_h, out, acc, colb, ones, zbuf, sem0, sem1):
    c = lax.axis_index("c")
    s = lax.axis_index("s")
    yoff = c * N

    t0 = pl.multiple_of(c * (EP // B) + s * BPT, BPT)
    pltpu.sync_copy(cols_h.at[pl.ds(t0, BPT)], colb)

    one16 = jnp.ones((16,), jnp.float32)

    def orow(i, carry):
        for j in range(CW // 16):
            ones[i, pl.ds(j * 16, 16)] = one16
        return carry

    lax.fori_loop(0, B, orow, 0)

    _zero_fill(zbuf, CW)
    _for_row_chunks(s, lambda r0: pltpu.sync_copy(zbuf, acc.at[pl.ds(r0, CH)]))
    plsc.subcore_barrier()

    def sstart(j, sem):
        pltpu.async_copy(ones, acc.at[colb.at[j]], sem, add=True)

    def sdrain(sem):
        pltpu.make_async_copy(out.at[pl.ds(0, B)], ones, sem).wait()

    sstart(0, sem0)

    def blk(g, carry):
        j0 = g * 2
        sstart(j0 + 1, sem1)
        sdrain(sem0)

        @pl.when(g < BPT // 2 - 1)
        def _():
            sstart(j0 + 2, sem0)

        sdrain(sem1)
        return carry

    lax.fori_loop(0, BPT // 2, blk, 0)
    plsc.subcore_barrier()

    def wb(r0):
        pltpu.sync_copy(acc.at[pl.ds(r0, CH)], zbuf)
        pltpu.sync_copy(zbuf, out.at[pl.ds(yoff + r0, CH)])

    _for_row_chunks(s, wb)


@functools.cache
def _get_counts():
    return pl.kernel(
        _counts_body,
        out_type=jax.ShapeDtypeStruct((2 * N, CW), jnp.float32),
        mesh=plsc.VectorSubcoreMesh(core_axis_name="c", subcore_axis_name="s"),
        scratch_types=[
            pltpu.VMEM_SHARED((NACC, CW), jnp.float32),
            pltpu.VMEM((BPT, B), jnp.int32),
            pltpu.VMEM((B, CW), jnp.float32),
            pltpu.VMEM((CH, CW), jnp.float32),
            pltpu.SemaphoreType.DMA,
            pltpu.SemaphoreType.DMA,
        ],
    )


_R = 1000
_CONTRACT = (((1,), (1,)), ((), ()))


def _tca_body(xa_ref, xp_ref, wl1_ref, wr1_ref, wr2_ref, bl1_ref, bl2_ref,
              y1_ref, z1_ref, z2_ref):
    xa = xa_ref[...]
    y1 = lax.dot_general(xa, wl1_ref[...], _CONTRACT,
                         preferred_element_type=jnp.float32)
    y1_ref[0] = y1[:, :HD]
    y1_ref[1] = y1[:, HD:]
    z1_ref[...] = lax.dot_general(xp_ref[...], wr1_ref[...], _CONTRACT,
                                  preferred_element_type=jnp.float32) + bl1_ref[...]
    z2_ref[...] = lax.dot_general(xa, wr2_ref[...], _CONTRACT,
                                  preferred_element_type=jnp.float32) + bl2_ref[...]


_tc_a = pl.pallas_call(
    _tca_body,
    grid=(N // _R,),
    in_specs=[
        pl.BlockSpec((_R, D), lambda i: (i, 0)),
        pl.BlockSpec((_R, D), lambda i: (i, 0)),
        pl.BlockSpec((D, D), lambda i: (0, 0)),
        pl.BlockSpec((D, D), lambda i: (0, 0)),
        pl.BlockSpec((D, D), lambda i: (0, 0)),
        pl.BlockSpec((1, D), lambda i: (0, 0)),
        pl.BlockSpec((1, D), lambda i: (0, 0)),
    ],
    out_specs=[
        pl.BlockSpec((2, _R, HD), lambda i: (0, i, 0)),
        pl.BlockSpec((_R, D), lambda i: (i, 0)),
        pl.BlockSpec((_R, D), lambda i: (i, 0)),
    ],
    out_shape=[
        jax.ShapeDtypeStruct((2, N, HD), jnp.float32),
        jax.ShapeDtypeStruct((N, D), jnp.float32),
        jax.ShapeDtypeStruct((N, D), jnp.float32),
    ],
)


def _tcb_body(s_ref, cnt_ref, z1_ref, wl2_ref, y2_ref):
    inv = 1.0 / jnp.maximum(cnt_ref[...], 1.0)
    h = jnp.concatenate([s_ref[0], s_ref[1]], axis=1) * inv + z1_ref[...]
    h = jnp.maximum(h, 0.0)
    y2 = lax.dot_general(h, wl2_ref[...], _CONTRACT,
                         preferred_element_type=jnp.float32)
    y2_ref[0] = y2[:, :HD]
    y2_ref[1] = y2[:, HD:]


_tc_b = pl.pallas_call(
    _tcb_body,
    grid=(N // _R,),
    in_specs=[
        pl.BlockSpec((2, _R, HD), lambda i: (0, i, 0)),
        pl.BlockSpec((_R, 1), lambda i: (i, 0)),
        pl.BlockSpec((_R, D), lambda i: (i, 0)),
        pl.BlockSpec((D, D), lambda i: (0, 0)),
    ],
    out_specs=pl.BlockSpec((2, _R, HD), lambda i: (0, i, 0)),
    out_shape=jax.ShapeDtypeStruct((2, N, HD), jnp.float32),
)


def _tcc_body(s_ref, cnt_ref, z2_ref, wo_ref, bo_ref, out_ref):
    inv = 1.0 / jnp.maximum(cnt_ref[...], 1.0)
    h = jnp.concatenate([s_ref[0], s_ref[1]], axis=1) * inv + z2_ref[...]
    h = jnp.maximum(h, 0.0)
    out_ref[...] = lax.dot_general(h, wo_ref[...], _CONTRACT,
                                   preferred_element_type=jnp.float32) + bo_ref[...]


_tc_c = pl.pallas_call(
    _tcc_body,
    grid=(N // _R,),
    in_specs=[
        pl.BlockSpec((2, _R, HD), lambda i: (0, i, 0)),
        pl.BlockSpec((_R, 1), lambda i: (i, 0)),
        pl.BlockSpec((_R, D), lambda i: (i, 0)),
        pl.BlockSpec((D, D), lambda i: (0, 0)),
        pl.BlockSpec((1, D), lambda i: (0, 0)),
    ],
    out_specs=pl.BlockSpec((_R, D), lambda i: (i, 0)),
    out_shape=jax.ShapeDtypeStruct((N, D), jnp.float32),
)


def _pad_idx(idx, fill):
    pad = jnp.full((EP - E,), fill, jnp.int32)
    return jnp.concatenate([idx, pad])


def kernel(x_author, x_paper, edge_index_writes, edge_index_written_by,
           Wl1, bl1, Wr1, Wl2, bl2, Wr2, Wo, bo):
    row1, col1 = edge_index_writes[0], edge_index_writes[1]
    row2, col2 = edge_index_written_by[0], edge_index_written_by[1]
    cols = jnp.concatenate([_pad_idx(col1, N), _pad_idx(col2, N)]
                           ).reshape(2 * EP // B, B)

    cntw = _get_counts()(cols)
    cnt1, cnt2 = cntw[:N, :1], cntw[N:, :1]

    y1s, z1, z2 = _tc_a(x_author, x_paper, Wl1, Wr1, Wr2,
                        bl1.reshape(1, D), bl2.reshape(1, D))
    s1 = _get_segsum()(y1s.reshape(2 * N, HD), row1, col1)
    y2s = _tc_b(s1.reshape(2, N, HD), cnt1, z1, Wl2)
    s2 = _get_segsum()(y2s.reshape(2 * N, HD), row2, col2)
    return _tc_c(s2.reshape(2, N, HD), cnt2, z2, Wo, bo.reshape(1, D))
